# Initial kernel scaffold; baseline (speedup 1.0000x reference)
import jax
import jax.numpy as jnp
from jax.experimental import pallas as pl


def kernel(x, adj, W1, b1, W2, b2, adj_lambda_1, adj_lambda_2, fc_w, fc_b):
    raise NotImplementedError("write your pallas kernel here")


# speedup vs baseline: 2.0165x; 2.0165x over previous
import numpy as np
import jax
import jax.numpy as jnp
from jax.experimental import pallas as pl

_N = 100
_LVLS = 3
_D0 = 16
_C1 = 16
_C2 = 32
_EDGE_P = 0.06


def _structure():
    rng = np.random.RandomState(0)
    A = rng.rand(_N, _N) < _EDGE_P
    A = np.triu(A, 1)
    A = A | A.T
    nbhd1 = [sorted(set([v]) | set(np.nonzero(A[v])[0].tolist()))
             for v in range(_N)]
    rf = [[[v] for v in range(_N)]]
    for _ in range(1, _LVLS):
        prev = rf[-1]
        cur = []
        for v in range(_N):
            s = set()
            for w in nbhd1[v]:
                s.update(prev[w])
            cur.append(sorted(s))
        rf.append(cur)
    return nbhd1, rf


_NBHD1, _RF = _structure()
_OUT_V = list(range(_LVLS))
_W_NEED = sorted(set().union(*[set(_NBHD1[v]) for v in _OUT_V]))
_K1 = {w: len(_RF[1][w]) for w in _W_NEED}

_T1 = sum(k * k for k in _K1.values())

_GXSEL = np.zeros((_N, _T1), np.float32)
_GR1 = np.zeros((_T1, _N), np.float32)
_GC1 = np.zeros((_T1, _N), np.float32)
_TOFF1 = {}
_to = 0
for _w in _W_NEED:
    _k = _K1[_w]
    _S = _RF[1][_w]
    _TOFF1[_w] = _to
    for _i in range(_k):
        for _j in range(_k):
            _t = _to + _i * _k + _j
            if _i == _j:
                _GXSEL[_S[_i], _t] = 1.0
            _GR1[_t, _S[_i]] = 1.0
            _GC1[_t, _S[_j]] = 1.0
    _to += _k * _k

_K2 = {v: len(_RF[2][v]) for v in _OUT_V}
_T2 = sum(K * K for K in _K2.values())

_KRON = {}
_GR2 = np.zeros((_T2, _N), np.float32)
_GC2 = np.zeros((_T2, _N), np.float32)
_TOFF2 = {}
_to2 = 0
for _v in _OUT_V:
    _K = _K2[_v]
    _S2 = _RF[2][_v]
    _TOFF2[_v] = _to2
    _blocks = []
    for _w in _NBHD1[_v]:
        _S1 = np.asarray(_RF[1][_w])
        _chi = (np.asarray(_S2)[:, None] == _S1[None, :]).astype(np.float32)
        _blocks.append(np.kron(_chi, _chi))
    _KRON[_v] = np.concatenate(_blocks, axis=1)
    for _I in range(_K):
        for _J in range(_K):
            _t = _to2 + _I * _K + _J
            _GR2[_t, _S2[_I]] = 1.0
            _GC2[_t, _S2[_J]] = 1.0
    _to2 += _K * _K

_FULLSEL = np.zeros((_C2, _T2), np.float32)
_bnd = []
for _v in _OUT_V:
    _KK = _K2[_v] * _K2[_v]
    _t0 = _TOFF2[_v]
    for _r in range(_KK):
        _ca = (_r * _C2) // _KK
        _cb = (_r * _C2 + _C2 - 1) // _KK
        if _ca == _cb:
            _FULLSEL[_ca, _t0 + _r] = 1.0
        else:
            _p = (_ca + 1) * _KK - _r * _C2
            _bnd.append((_t0 + _r, _ca, _p))
_NB = max(len(_bnd), 1)
_PB = np.zeros((_NB, _T2), np.float32)
_MLOW = np.zeros((_NB, _C2), np.float32)
_MHIGH = np.zeros((_NB, _C2), np.float32)
_MA = np.zeros((_C2, _NB), np.float32)
_MBH = np.zeros((_C2, _NB), np.float32)
for _b, (_rg, _ca, _p) in enumerate(_bnd):
    _PB[_b, _rg] = 1.0
    _MLOW[_b, :_p] = 1.0
    _MHIGH[_b, _p:] = 1.0
    _MA[_ca, _b] = 1.0
    _MBH[_ca + 1, _b] = 1.0

_L1_SLICES = [(_TOFF1[w], _K1[w] * _K1[w]) for w in _W_NEED]
_L2_SLICES = [(_TOFF2[v], _K2[v] * _K2[v]) for v in _OUT_V]
_HOFF = {}
_NBH_SLICES = {}


def _k1_kernel(x_ref, adj_ref, lam1_ref, gxsel_ref, gr1_ref, gc1_ref,
               gr2_ref, gc2_ref, aclT_ref, avec2_ref):
    f32 = jnp.float32

    def mm(a, b):
        return jax.lax.dot_general(a, b, (((1,), (0,)), ((), ())),
                                   preferred_element_type=f32)

    adj = adj_ref[...]
    lam1 = lam1_ref[0, 0]
    xsel = jax.lax.dot_general(x_ref[...], gxsel_ref[...],
                               (((0,), (0,)), ((), ())),
                               preferred_element_type=f32)
    avec1 = jnp.sum(mm(gr1_ref[...], adj) * gc1_ref[...],
                    axis=1, keepdims=True)
    aclT_ref[...] = xsel + lam1 * jnp.broadcast_to(avec1.T, (_D0, _T1))
    avec2_ref[...] = jnp.sum(mm(gr2_ref[...], adj) * gc2_ref[...],
                             axis=1, keepdims=True)


def _k2_kernel(flat1_ref, w1_ref, b1_ref, h1_ref):
    pre = jax.lax.dot_general(flat1_ref[...], w1_ref[...],
                              (((1,), (1,)), ((), ())),
                              preferred_element_type=jnp.float32)
    h1_ref[...] = jnp.maximum(pre + b1_ref[...], 0.0)


def _k3_kernel(featsT_ref, avec2_ref, lam2_ref,
               kron0_ref, kron1_ref, kron2_ref, aggT_ref):
    f32 = jnp.float32

    def mm(a, b):
        return jax.lax.dot_general(a, b, (((1,), (0,)), ((), ())),
                                   preferred_element_type=f32)

    lam2 = lam2_ref[0, 0]
    feats_cl = featsT_ref[...].T
    avec2 = avec2_ref[...]
    kron_refs = {0: kron0_ref, 1: kron1_ref, 2: kron2_ref}
    parts = []
    for v in _OUT_V:
        K = _K2[v]
        t0 = _TOFF2[v]
        H_v = jnp.concatenate(
            [feats_cl[o:o + n, :] for (o, n) in _NBH_SLICES[v]], axis=0)
        parts.append(mm(kron_refs[v][...], H_v)
                     + lam2 * avec2[t0:t0 + K * K, :])
    aggT_ref[...] = jnp.concatenate(parts, axis=0).T


def _k4_kernel(flat2_ref, w2_ref, b2_ref, fcw_ref, fcb_ref,
               fullsel_ref, pb_ref, mlow_ref, mhigh_ref, ma_ref, mbh_ref,
               out_ref, g_ref):
    f32 = jnp.float32

    def mm(a, b):
        return jax.lax.dot_general(a, b, (((1,), (0,)), ((), ())),
                                   preferred_element_type=f32)

    def mmt(a, b):
        return jax.lax.dot_general(a, b, (((1,), (1,)), ((), ())),
                                   preferred_element_type=f32)

    pre = mmt(flat2_ref[...], w2_ref[...])
    h2 = jnp.maximum(pre + b2_ref[...], 0.0)
    rowtot = jnp.sum(h2, axis=1, keepdims=True)
    s = mm(fullsel_ref[...], rowtot)
    hb = mm(pb_ref[...], h2)
    s = s + mm(ma_ref[...],
               jnp.sum(hb * mlow_ref[...], axis=1, keepdims=True))
    s = s + mm(mbh_ref[...],
               jnp.sum(hb * mhigh_ref[...], axis=1, keepdims=True))
    g_ref[...] = s
    out_ref[...] = mm(fcw_ref[...], s) + fcb_ref[...]


def kernel(x, adj, W1, b1, W2, b2, adj_lambda_1, adj_lambda_2, fc_w, fc_b):
    f32 = jnp.float32
    aclT, avec2 = pl.pallas_call(
        _k1_kernel,
        out_shape=[jax.ShapeDtypeStruct((_D0, _T1), f32),
                   jax.ShapeDtypeStruct((_T2, 1), f32)],
    )(x, adj, adj_lambda_1.reshape(1, 1), jnp.asarray(_GXSEL),
      jnp.asarray(_GR1), jnp.asarray(_GC1),
      jnp.asarray(_GR2), jnp.asarray(_GC2))

    flat1 = jnp.concatenate(
        [aclT[:, o:o + n].reshape(n, _D0) for (o, n) in _L1_SLICES], axis=0)

    h1 = pl.pallas_call(
        _k2_kernel,
        out_shape=jax.ShapeDtypeStruct((_T1, _C1), f32),
    )(flat1, W1, b1.reshape(1, _C1))

    featsT = jnp.concatenate(
        [h1[o:o + n, :].reshape(_C1, n) for (o, n) in _L1_SLICES], axis=1)

    aggT = pl.pallas_call(
        _k3_kernel,
        out_shape=jax.ShapeDtypeStruct((_C1, _T2), f32),
    )(featsT, avec2, adj_lambda_2.reshape(1, 1),
      jnp.asarray(_KRON[0]), jnp.asarray(_KRON[1]), jnp.asarray(_KRON[2]))

    flat2 = jnp.concatenate(
        [aggT[:, o:o + n].reshape(n, _C1) for (o, n) in _L2_SLICES], axis=0)

    out, g_col = pl.pallas_call(
        _k4_kernel,
        out_shape=[jax.ShapeDtypeStruct((1, 1), f32),
                   jax.ShapeDtypeStruct((_C2, 1), f32)],
    )(flat2, W2, b2.reshape(1, _C2), fc_w, fc_b.reshape(1, 1),
      jnp.asarray(_FULLSEL), jnp.asarray(_PB), jnp.asarray(_MLOW),
      jnp.asarray(_MHIGH), jnp.asarray(_MA), jnp.asarray(_MBH))
    return out, g_col.reshape(1, _C2)


for _w in _W_NEED:
    _HOFF[_w] = _TOFF1[_w]
for _v in _OUT_V:
    _NBH_SLICES[_v] = [(_HOFF[_w], _K1[_w] * _K1[_w]) for _w in _NBHD1[_v]]


# speedup vs baseline: 2.7394x; 1.3585x over previous
import functools
import numpy as np
import jax
import jax.numpy as jnp
from jax import lax
from jax.experimental import pallas as pl
from jax.experimental.pallas import tpu as pltpu
from jax.experimental.pallas import tpu_sc as plsc

_N = 100
_LVLS = 3
_D0 = 16
_C1 = 16
_C2 = 32
_EDGE_P = 0.06


def _structure():
    rng = np.random.RandomState(0)
    A = rng.rand(_N, _N) < _EDGE_P
    A = np.triu(A, 1)
    A = A | A.T
    nbhd1 = [sorted(set([v]) | set(np.nonzero(A[v])[0].tolist()))
             for v in range(_N)]
    rf = [[[v] for v in range(_N)]]
    for _ in range(1, _LVLS):
        prev = rf[-1]
        cur = []
        for v in range(_N):
            s = set()
            for w in nbhd1[v]:
                s.update(prev[w])
            cur.append(sorted(s))
        rf.append(cur)
    return nbhd1, rf


_NBHD1, _RF = _structure()
_OUT_V = list(range(_LVLS))
_W_NEED = sorted(set().union(*[set(_NBHD1[v]) for v in _OUT_V]))
_K1 = {w: len(_RF[1][w]) for w in _W_NEED}
_K2 = {v: len(_RF[2][v]) for v in _OUT_V}

_T1 = sum(k * k for k in _K1.values())
_T2 = sum(K * K for K in _K2.values())
_NW = 32
_R1W = 96
_R2W = 96
_T1P = 16 * _R1W
_T2P = _NW * _R2W
_E1 = _T1P * 16
_E1W = _R1W * 16
_E2W = _R2W * 16
_FAN = 7
_SENT = _E1

_TOFF1 = {}
_o = 0
for _w in _W_NEED:
    _TOFF1[_w] = _o
    _o += _K1[_w] * _K1[_w]
_TOFF2 = {}
_o = 0
for _v in _OUT_V:
    _TOFF2[_v] = _o
    _o += _K2[_v] * _K2[_v]

_XI = np.zeros((_E1,), np.int32)
_XMF = np.zeros((_E1,), np.float32)
_AI = np.zeros((_E1,), np.int32)
for _w in _W_NEED:
    _k = _K1[_w]
    _S = _RF[1][_w]
    _base = _TOFF1[_w] * 16
    for _m in range(16 * _k * _k):
        _e = _base + _m
        _c, _rem = divmod(_m, _k * _k)
        _i, _j = divmod(_rem, _k)
        _AI[_e] = _S[_i] * _N + _S[_j]
        if _i == _j:
            _XI[_e] = _S[_i] * 16 + _c
            _XMF[_e] = 1.0

_A2I = np.zeros((_T2P * 16,), np.int32)
_SRC = np.full((_FAN, _T2P * 16), _SENT, np.int32)
_CNT = np.zeros((_T2P * 16,), np.int32)
for _v in _OUT_V:
    _K = _K2[_v]
    _S2 = _RF[2][_v]
    _pos2 = {u: i for i, u in enumerate(_S2)}
    _b2 = _TOFF2[_v] * 16
    for _m in range(16 * _K * _K):
        _e = _b2 + _m
        _c, _rem = divmod(_m, _K * _K)
        _I, _J = divmod(_rem, _K)
        _A2I[_e] = _S2[_I] * _N + _S2[_J]
    for _w in _NBHD1[_v]:
        _k = _K1[_w]
        _S1 = _RF[1][_w]
        for _c in range(16):
            for _il in range(_k):
                for _jl in range(_k):
                    _m = _c * _K * _K + _pos2[_S1[_il]] * _K + _pos2[_S1[_jl]]
                    _e = _b2 + _m
                    _SRC[_CNT[_e], _e] = (_TOFF1[_w] * 16
                                          + _c * _k * _k + _il * _k + _jl)
                    _CNT[_e] += 1

_CH2 = np.full((_T2P, 32), 32, np.int32)
for _v in _OUT_V:
    _K = _K2[_v]
    for _rl in range(_K * _K):
        _row = _TOFF2[_v] + _rl
        for _oo in range(32):
            _CH2[_row, _oo] = (_rl * 32 + _oo) // (_K * _K)

_XI_O = 0
_XM_O = _E1W
_AI_O = 2 * _E1W
_A2_O = 3 * _E1W
_SR_O = 4 * _E1W
_CH_O = _SR_O + _FAN * _E2W
_RTBL = _CH_O + _R2W * 32

_TBL = np.zeros((_NW, _RTBL), np.int32)
for _wid in range(_NW):
    _sid = _wid // 2
    _sl1 = slice(_sid * _E1W, (_sid + 1) * _E1W)
    _sl2 = slice(_wid * _E2W, (_wid + 1) * _E2W)
    _TBL[_wid, _XI_O:_XI_O + _E1W] = _XI[_sl1]
    _TBL[_wid, _XM_O:_XM_O + _E1W] = _XMF[_sl1].view(np.int32)
    _TBL[_wid, _AI_O:_AI_O + _E1W] = _AI[_sl1]
    _TBL[_wid, _A2_O:_A2_O + _E2W] = _A2I[_sl2]
    for _f in range(_FAN):
        _TBL[_wid, _SR_O + _f * _E2W:_SR_O + (_f + 1) * _E2W] = _SRC[_f, _sl2]
    _cht = np.empty((_R2W // 16, 32, 16), np.int32)
    for _b in range(_R2W // 16):
        for _oo in range(32):
            for _i in range(16):
                _cht[_b, _oo, _i] = _CH2[_wid * _R2W + _b * 16 + _i, _oo]
    _TBL[_wid, _CH_O:_CH_O + _R2W * 32] = _cht.ravel()

_P_MISC = 0
_P_B1 = 16
_P_B2 = 32
_P_FCW = 64
_P_W1 = 96
_P_W2 = 352
_NPAR = 864

_f32 = jnp.float32


def _sc_body(x_hbm, adj_hbm, p_hbm, tbl_hbm, s_out,
             xv, adjv, pv, tblv, h1c, h1ext, fl2, sacc, sh_h1):
    cid = lax.axis_index("c")
    sid = lax.axis_index("s")
    wid = sid * 2 + cid
    iot = lax.iota(jnp.int32, 16)

    pltpu.sync_copy(x_hbm, xv)
    pltpu.sync_copy(adj_hbm, adjv)
    pltpu.sync_copy(p_hbm, pv)
    pltpu.sync_copy(tbl_hbm.at[wid], tblv)

    misc = pv[pl.ds(_P_MISC, 16)]
    lam1 = misc[0]
    lam2 = misc[1]
    b1v = pv[pl.ds(_P_B1, 16)]
    w1v = [pv[pl.ds(_P_W1 + o * 16, 16)] for o in range(16)]

    def phase_a(b, _):
        base = b * 256
        cols = []
        for c in range(16):
            ei = base + c + iot * 16
            xi = plsc.load_gather(tblv, [_XI_O + ei])
            xm = plsc.bitcast(plsc.load_gather(tblv, [_XM_O + ei]), _f32)
            ai = plsc.load_gather(tblv, [_AI_O + ei])
            xval = plsc.load_gather(xv, [xi])
            aval = plsc.load_gather(adjv, [ai])
            cols.append(xm * xval + lam1 * aval)
        for o in range(16):
            acc = cols[0] * w1v[o][0]
            for c in range(1, 16):
                acc = acc + cols[c] * w1v[o][c]
            val = jnp.maximum(acc + b1v[o], 0.0)
            plsc.store_scatter(h1c, [base + o + iot * 16], val)
        return _

    lax.fori_loop(0, _R1W // 16, phase_a, None)

    pltpu.sync_copy(h1c, sh_h1.at[pl.ds(sid * _E1W, _E1W)])
    plsc.subcore_barrier()
    pltpu.sync_copy(sh_h1, h1ext.at[pl.ds(0, _E1)])
    h1ext[pl.ds(_E1, 16)] = jnp.zeros((16,), _f32)

    def phase_b(g, _):
        e = g * 16 + iot
        a2 = plsc.load_gather(tblv, [_A2_O + e])
        acc = lam2 * plsc.load_gather(adjv, [a2])
        for f in range(_FAN):
            si = plsc.load_gather(tblv, [_SR_O + f * _E2W + e])
            acc = acc + plsc.load_gather(h1ext, [si])
        plsc.store_scatter(fl2, [e], acc)
        return _

    lax.fori_loop(0, _R2W, phase_b, None)

    sacc[pl.ds(0, 16)] = jnp.zeros((16,), _f32)
    sacc[pl.ds(16, 16)] = jnp.zeros((16,), _f32)
    sacc[pl.ds(32, 16)] = jnp.zeros((16,), _f32)

    b2v = [pv[pl.ds(_P_B2, 16)], pv[pl.ds(_P_B2 + 16, 16)]]
    w2v = [pv[pl.ds(_P_W2 + o * 16, 16)] for o in range(32)]

    def phase_c(b, _):
        base = b * 256
        cols = []
        for c in range(16):
            cols.append(plsc.load_gather(fl2, [base + c + iot * 16]))
        for o in range(32):
            acc = cols[0] * w2v[o][0]
            for c in range(1, 16):
                acc = acc + cols[c] * w2v[o][c]
            val = jnp.maximum(acc + b2v[o // 16][o % 16], 0.0)
            sidx = plsc.load_gather(tblv, [_CH_O + b * 512 + o * 16 + iot])
            plsc.addupdate_scatter(sacc, [sidx], val)
        return _

    lax.fori_loop(0, _R2W // 16, phase_c, None)

    pltpu.sync_copy(sacc, s_out.at[wid])


def _tc_reduce(sp_ref, fcw_ref, fcb_ref, out_ref, g_ref):
    sp = sp_ref[...]
    stot = jnp.sum(sp, axis=0, keepdims=True)
    g_row = stot[:, 0:_C2]
    g_ref[...] = g_row
    prod = g_row * fcw_ref[...]
    out_ref[...] = jnp.sum(prod, axis=1, keepdims=True) + fcb_ref[...]


def kernel(x, adj, W1, b1, W2, b2, adj_lambda_1, adj_lambda_2, fc_w, fc_b):
    params = jnp.concatenate([
        adj_lambda_1.reshape(-1), adj_lambda_2.reshape(-1),
        fc_b.reshape(-1), jnp.zeros((13,), _f32),
        b1.reshape(-1), b2.reshape(-1), fc_w.reshape(-1),
        W1.reshape(-1), W2.reshape(-1),
    ])
    mesh = plsc.VectorSubcoreMesh(core_axis_name="c", subcore_axis_name="s")
    sc = functools.partial(
        pl.kernel, _sc_body, mesh=mesh,
        compiler_params=pltpu.CompilerParams(needs_layout_passes=False),
        out_type=jax.ShapeDtypeStruct((_NW, 48), _f32),
        scratch_types=[
            pltpu.VMEM((1600,), _f32),
            pltpu.VMEM((_N * _N,), _f32),
            pltpu.VMEM((_NPAR,), _f32),
            pltpu.VMEM((_RTBL,), jnp.int32),
            pltpu.VMEM((_E1W,), _f32),
            pltpu.VMEM((_E1 + 16,), _f32),
            pltpu.VMEM((_E2W,), _f32),
            pltpu.VMEM((48,), _f32),
            pltpu.VMEM_SHARED((_E1,), _f32),
        ],
    )()
    s_part = sc(x.reshape(-1), adj.reshape(-1), params, jnp.asarray(_TBL))

    out, g = pl.pallas_call(
        _tc_reduce,
        out_shape=[jax.ShapeDtypeStruct((1, 1), _f32),
                   jax.ShapeDtypeStruct((1, _C2), _f32)],
    )(s_part, fc_w, fc_b.reshape(1, 1))
    return out, g
